# baseline (speedup 1.0000x reference)
import jax
import jax.numpy as jnp
from jax.experimental import pallas as pl
from jax.experimental.pallas import tpu as pltpu

_BLOCK_ROWS = 2048


def _add_row_kernel(idx_ref, emb_ref, x_ref, o_ref):
    row = emb_ref[idx_ref[0], :]
    o_ref[...] = x_ref[...] + row[None, :]


def kernel(x, rate_id, rate_embedding):
    b, s, d = x.shape
    rows = b * s
    x2 = x.reshape(rows, d)
    block = min(_BLOCK_ROWS, rows)
    idx = jnp.asarray([rate_id], dtype=jnp.int32)
    out = pl.pallas_call(
        _add_row_kernel,
        grid_spec=pltpu.PrefetchScalarGridSpec(
            num_scalar_prefetch=1,
            grid=(rows // block,),
            in_specs=[
                pl.BlockSpec(rate_embedding.shape, lambda i, idx_ref: (0, 0)),
                pl.BlockSpec((block, d), lambda i, idx_ref: (i, 0)),
            ],
            out_specs=pl.BlockSpec((block, d), lambda i, idx_ref: (i, 0)),
        ),
        out_shape=jax.ShapeDtypeStruct((rows, d), x.dtype),
        compiler_params=pltpu.CompilerParams(
            dimension_semantics=("parallel",),
        ),
    )(idx, rate_embedding, x2)
    return out.reshape(b, s, d)
